# Initial kernel scaffold; baseline (speedup 1.0000x reference)
import jax
import jax.numpy as jnp
from jax.experimental import pallas as pl


def kernel(x, W):
    raise NotImplementedError("write your pallas kernel here")


# speedup vs baseline: 5.9822x; 5.9822x over previous
import functools

import jax
import jax.numpy as jnp
from jax import lax
from jax.experimental import pallas as pl
from jax.experimental.pallas import tpu as pltpu
from jax.experimental.pallas import tpu_sc as plsc

_K = 8192
_D = 256
_N = 8192
_T = 256
_KC = 1024
_NT = _N // _T
_NKC = _K // _KC
_COMMIT = 0.25


def _vq_argmin_body(z_ref, w_ref, zsq_ref, idx_ref, enc_ref, lp_ref):
    z = z_ref[...]
    zsq = zsq_ref[0, 0, :]

    def chunk(k, carry):
        mv, mi = carry
        w = w_ref[pl.ds(k * _KC, _KC), :]
        m = lax.dot_general(z, w, (((1,), (1,)), ((), ())),
                            preferred_element_type=jnp.float32)
        wsq = jnp.sum(w * w, axis=1)
        d = (zsq[:, None] + wsq[None, :]) - 2.0 * m
        bm = jnp.min(d, axis=1)
        cols = lax.broadcasted_iota(jnp.int32, (_T, _KC), 1) + k * _KC
        ba = jnp.min(jnp.where(d == bm[:, None], cols, jnp.int32(2**30)),
                     axis=1)
        upd = bm < mv
        return jnp.where(upd, bm, mv), jnp.where(upd, ba, mi)

    def half_min(k0):
        mv0 = jnp.full((_T,), jnp.inf, jnp.float32)
        mi0 = jnp.zeros((_T,), jnp.int32)
        return lax.fori_loop(k0, k0 + _NKC // 2, chunk, (mv0, mi0))

    mv_a, mi_a = half_min(0)
    mv_b, mi_b = half_min(_NKC // 2)
    mva_q = mv_a.astype(jnp.bfloat16).astype(jnp.float32)
    upd = mv_b < mva_q
    mv = jnp.where(upd, mv_b, mv_a)
    mi = jnp.where(upd, mi_b, mi_a)
    idx_ref[0, 0, :] = mi
    cols = lax.broadcasted_iota(jnp.int32, (_T, _K), 1)
    enc_ref[...] = (cols == mi[:, None]).astype(jnp.float32)
    lp_ref[0, 0, 0] = jnp.sum(mv)


def _vq_argmin(flat, W, zsq3):
    return pl.pallas_call(
        _vq_argmin_body,
        grid=(_NT,),
        in_specs=[
            pl.BlockSpec((_T, _D), lambda t: (t, 0)),
            pl.BlockSpec((_K, _D), lambda t: (0, 0)),
            pl.BlockSpec((1, 1, _T), lambda t: (t, 0, 0)),
        ],
        out_specs=[
            pl.BlockSpec((1, 1, _T), lambda t: (t, 0, 0)),
            pl.BlockSpec((_T, _K), lambda t: (t, 0)),
            pl.BlockSpec((1, 1, 1), lambda t: (t, 0, 0),
                         memory_space=pltpu.SMEM),
        ],
        out_shape=[
            jax.ShapeDtypeStruct((_NT, 1, _T), jnp.int32),
            jax.ShapeDtypeStruct((_N, _K), jnp.float32),
            jax.ShapeDtypeStruct((_NT, 1, 1), jnp.float32),
        ],
    )(flat, W, zsq3)


def _gather_rows(W, idx3):
    info = plsc.get_sparse_core_info()
    nc, ns = info.num_cores, info.num_subcores
    nw = nc * ns
    bpw = _N // nw
    nch = bpw // 128
    mesh = plsc.VectorSubcoreMesh(core_axis_name="c", subcore_axis_name="s")

    @functools.partial(
        pl.kernel, mesh=mesh,
        out_type=jax.ShapeDtypeStruct((_N, _D), jnp.float32),
        scratch_types=[
            pltpu.VMEM((nch, 128), jnp.int32),
            pltpu.VMEM((bpw, _D), jnp.float32),
            pltpu.SemaphoreType.DMA,
        ],
    )
    def body(idx_hbm, w_hbm, out_hbm, idx_v, rows_v, sem):
        wid = lax.axis_index("s") * nc + lax.axis_index("c")
        base = wid * bpw
        pltpu.sync_copy(idx_hbm.at[wid], idx_v)
        copies = [
            pltpu.async_copy(w_hbm.at[idx_v.at[j]],
                             rows_v.at[pl.ds(j * 128, 128)], sem)
            for j in range(nch)
        ]
        for c in copies:
            c.wait()
        pltpu.sync_copy(rows_v, out_hbm.at[pl.ds(base, bpw)])

    return body(idx3, W)


def kernel(x, W):
    inputs = jnp.transpose(x, (0, 2, 3, 1))
    flat = inputs.reshape(-1, _D)
    zsq = jnp.sum(flat ** 2, axis=1)
    idx3, enc, lpart = _vq_argmin(flat, W, zsq.reshape(_NT, 1, _T))
    nw = 32
    qflat = _gather_rows(W, idx3.reshape(nw, _N // nw // 128, 128))
    quantized = qflat.reshape(inputs.shape)
    l = jnp.sum(lpart) / (_N * _D)
    loss = l + _COMMIT * l
    return loss, jnp.transpose(quantized, (0, 3, 1, 2)), enc


# speedup vs baseline: 6.0557x; 1.0123x over previous
import functools

import jax
import jax.numpy as jnp
from jax import lax
from jax.experimental import pallas as pl
from jax.experimental.pallas import tpu as pltpu
from jax.experimental.pallas import tpu_sc as plsc

_K = 8192
_D = 256
_N = 8192
_T = 256
_KC = 1024
_NT = _N // _T
_NKC = _K // _KC
_COMMIT = 0.25


def _vq_argmin_body(z_ref, w2_ref, zsq_ref, idx_ref, enc_ref, lp_ref):
    z = z_ref[...]
    zsq = zsq_ref[0, 0, :]
    cols0 = lax.broadcasted_iota(jnp.int32, (_T, _KC), 1)

    def chunk(k, carry):
        mv, mi = carry
        w2 = w2_ref[pl.ds(k * _KC, _KC), :]
        m2 = lax.dot_general(z, w2, (((1,), (1,)), ((), ())),
                             preferred_element_type=jnp.float32)
        wsq = jnp.sum(w2 * w2, axis=1) * 0.25
        d = (zsq[:, None] + wsq[None, :]) - m2
        bm = jnp.min(d, axis=1)
        ba = jnp.min(jnp.where(d == bm[:, None], cols0, jnp.int32(2**30)),
                     axis=1) + k * _KC
        upd = bm < mv
        return jnp.where(upd, bm, mv), jnp.where(upd, ba, mi)

    def half_min(k0):
        mv0 = jnp.full((_T,), jnp.inf, jnp.float32)
        mi0 = jnp.zeros((_T,), jnp.int32)
        return lax.fori_loop(k0, k0 + _NKC // 2, chunk, (mv0, mi0))

    mv_a, mi_a = half_min(0)
    mv_b, mi_b = half_min(_NKC // 2)
    mva_q = mv_a.astype(jnp.bfloat16).astype(jnp.float32)
    upd = mv_b < mva_q
    mv = jnp.where(upd, mv_b, mv_a)
    mi = jnp.where(upd, mi_b, mi_a)
    idx_ref[0, 0, :] = mi
    cols = lax.broadcasted_iota(jnp.int32, (_T, _K), 1)
    enc_ref[...] = (cols == mi[:, None]).astype(jnp.float32)
    lp_ref[0, 0, 0] = jnp.sum(mv)


def _vq_argmin(flat, W, zsq3):
    return pl.pallas_call(
        _vq_argmin_body,
        grid=(_NT,),
        in_specs=[
            pl.BlockSpec((_T, _D), lambda t: (t, 0)),
            pl.BlockSpec((_K, _D), lambda t: (0, 0)),
            pl.BlockSpec((1, 1, _T), lambda t: (t, 0, 0)),
        ],
        out_specs=[
            pl.BlockSpec((1, 1, _T), lambda t: (t, 0, 0)),
            pl.BlockSpec((_T, _K), lambda t: (t, 0)),
            pl.BlockSpec((1, 1, 1), lambda t: (t, 0, 0),
                         memory_space=pltpu.SMEM),
        ],
        out_shape=[
            jax.ShapeDtypeStruct((_NT, 1, _T), jnp.int32),
            jax.ShapeDtypeStruct((_N, _K), jnp.float32),
            jax.ShapeDtypeStruct((_NT, 1, 1), jnp.float32),
        ],
    )(flat, W, zsq3)


def _gather_rows(W, idx3):
    info = plsc.get_sparse_core_info()
    nc, ns = info.num_cores, info.num_subcores
    nw = nc * ns
    bpw = _N // nw
    nch = bpw // 128
    mesh = plsc.VectorSubcoreMesh(core_axis_name="c", subcore_axis_name="s")

    @functools.partial(
        pl.kernel, mesh=mesh,
        out_type=jax.ShapeDtypeStruct((_N, _D), jnp.float32),
        scratch_types=[
            pltpu.VMEM((nch, 128), jnp.int32),
            pltpu.VMEM((bpw, _D), jnp.float32),
            pltpu.SemaphoreType.DMA,
        ],
    )
    def body(idx_hbm, w_hbm, out_hbm, idx_v, rows_v, sem):
        wid = lax.axis_index("s") * nc + lax.axis_index("c")
        base = wid * bpw
        pltpu.sync_copy(idx_hbm.at[wid], idx_v)
        copies = [
            pltpu.async_copy(w_hbm.at[idx_v.at[j]],
                             rows_v.at[pl.ds(j * 128, 128)], sem)
            for j in range(nch)
        ]
        for c in copies:
            c.wait()
        pltpu.sync_copy(rows_v, out_hbm.at[pl.ds(base, bpw)])

    return body(idx3, W)


def kernel(x, W):
    inputs = jnp.transpose(x, (0, 2, 3, 1))
    flat = inputs.reshape(-1, _D)
    zsq = jnp.sum(flat ** 2, axis=1)
    idx3, enc, lpart = _vq_argmin(flat, W * 2.0, zsq.reshape(_NT, 1, _T))
    nw = 32
    qflat = _gather_rows(W, idx3.reshape(nw, _N // nw // 128, 128))
    quantized = qflat.reshape(inputs.shape)
    l = jnp.sum(lpart) / (_N * _D)
    loss = l + _COMMIT * l
    return loss, jnp.transpose(quantized, (0, 3, 1, 2)), enc


# speedup vs baseline: 6.2472x; 1.0316x over previous
import functools

import jax
import jax.numpy as jnp
from jax import lax
from jax.experimental import pallas as pl
from jax.experimental.pallas import tpu as pltpu
from jax.experimental.pallas import tpu_sc as plsc

_K = 8192
_D = 256
_N = 8192
_T = 256
_KC = 1024
_NT = _N // _T
_NKC = _K // _KC
_COMMIT = 0.25


def _vq_argmin_body(z_ref, w2_ref, zsq_ref, idx_ref, enc_ref, lp_ref):
    z = z_ref[...]
    zsq = zsq_ref[0, 0, :]
    cols0 = lax.broadcasted_iota(jnp.int32, (_T, _KC), 1)

    def chunk(k, carry):
        mv, mi = carry
        w2 = w2_ref[pl.ds(k * _KC, _KC), :]
        m2 = lax.dot_general(z, w2, (((1,), (1,)), ((), ())),
                             preferred_element_type=jnp.float32)
        wsq = jnp.sum(w2 * w2, axis=1) * 0.25
        d = (zsq[:, None] + wsq[None, :]) - m2
        bm = jnp.min(d, axis=1)
        ba = jnp.min(jnp.where(d == bm[:, None], cols0, jnp.int32(2**30)),
                     axis=1) + k * _KC
        upd = bm < mv
        return jnp.where(upd, bm, mv), jnp.where(upd, ba, mi)

    def half_min(k0):
        mv = jnp.full((_T,), jnp.inf, jnp.float32)
        mi = jnp.zeros((_T,), jnp.int32)
        for k in range(k0, k0 + _NKC // 2):
            mv, mi = chunk(k, (mv, mi))
        return mv, mi

    mv_a, mi_a = half_min(0)
    mv_b, mi_b = half_min(_NKC // 2)
    mva_q = mv_a.astype(jnp.bfloat16).astype(jnp.float32)
    upd = mv_b < mva_q
    mv = jnp.where(upd, mv_b, mv_a)
    mi = jnp.where(upd, mi_b, mi_a)
    idx_ref[0, 0, :] = mi
    cols = lax.broadcasted_iota(jnp.int32, (_T, _K), 1)
    enc_ref[...] = (cols == mi[:, None]).astype(jnp.float32)
    lp_ref[0, 0, 0] = jnp.sum(mv)


def _vq_argmin(flat, W, zsq3):
    return pl.pallas_call(
        _vq_argmin_body,
        grid=(_NT,),
        in_specs=[
            pl.BlockSpec((_T, _D), lambda t: (t, 0)),
            pl.BlockSpec((_K, _D), lambda t: (0, 0)),
            pl.BlockSpec((1, 1, _T), lambda t: (t, 0, 0)),
        ],
        out_specs=[
            pl.BlockSpec((1, 1, _T), lambda t: (t, 0, 0)),
            pl.BlockSpec((_T, _K), lambda t: (t, 0)),
            pl.BlockSpec((1, 1, 1), lambda t: (t, 0, 0),
                         memory_space=pltpu.SMEM),
        ],
        out_shape=[
            jax.ShapeDtypeStruct((_NT, 1, _T), jnp.int32),
            jax.ShapeDtypeStruct((_N, _K), jnp.float32),
            jax.ShapeDtypeStruct((_NT, 1, 1), jnp.float32),
        ],
    )(flat, W, zsq3)


def _gather_rows(W, idx3):
    info = plsc.get_sparse_core_info()
    nc, ns = info.num_cores, info.num_subcores
    nw = nc * ns
    bpw = _N // nw
    nch = bpw // 128
    mesh = plsc.VectorSubcoreMesh(core_axis_name="c", subcore_axis_name="s")

    @functools.partial(
        pl.kernel, mesh=mesh,
        out_type=jax.ShapeDtypeStruct((_N, _D), jnp.float32),
        scratch_types=[
            pltpu.VMEM((nch, 128), jnp.int32),
            pltpu.VMEM((bpw, _D), jnp.float32),
            pltpu.SemaphoreType.DMA,
        ],
    )
    def body(idx_hbm, w_hbm, out_hbm, idx_v, rows_v, sem):
        wid = lax.axis_index("s") * nc + lax.axis_index("c")
        base = wid * bpw
        pltpu.sync_copy(idx_hbm.at[wid], idx_v)
        copies = [
            pltpu.async_copy(w_hbm.at[idx_v.at[j]],
                             rows_v.at[pl.ds(j * 128, 128)], sem)
            for j in range(nch)
        ]
        for c in copies:
            c.wait()
        pltpu.sync_copy(rows_v, out_hbm.at[pl.ds(base, bpw)])

    return body(idx3, W)


def kernel(x, W):
    inputs = jnp.transpose(x, (0, 2, 3, 1))
    flat = inputs.reshape(-1, _D)
    zsq = jnp.sum(flat ** 2, axis=1)
    idx3, enc, lpart = _vq_argmin(flat, W * 2.0, zsq.reshape(_NT, 1, _T))
    nw = 32
    qflat = _gather_rows(W, idx3.reshape(nw, _N // nw // 128, 128))
    quantized = qflat.reshape(inputs.shape)
    l = jnp.sum(lpart) / (_N * _D)
    loss = l + _COMMIT * l
    return loss, jnp.transpose(quantized, (0, 3, 1, 2)), enc


# speedup vs baseline: 6.6236x; 1.0603x over previous
import functools

import jax
import jax.numpy as jnp
from jax import lax
from jax.experimental import pallas as pl
from jax.experimental.pallas import tpu as pltpu
from jax.experimental.pallas import tpu_sc as plsc

_K = 8192
_D = 256
_N = 8192
_T = 256
_KC = 1024
_NT = _N // _T
_NKC = _K // _KC
_COMMIT = 0.25


def _vq_argmin_body(z_ref, w2_ref, zsq_ref, idx_ref, enc_ref, lp_ref):
    z = z_ref[...]
    zsq = zsq_ref[0, 0, :]
    cols0 = lax.broadcasted_iota(jnp.int32, (_T, _KC), 1)

    def chunk(k, carry):
        mv, mi = carry
        w2 = w2_ref[pl.ds(k * _KC, _KC), :]
        m2 = lax.dot_general(z, w2, (((1,), (1,)), ((), ())),
                             preferred_element_type=jnp.float32)
        wsq = jnp.sum(w2 * w2, axis=1) * 0.25
        d = (zsq[:, None] + wsq[None, :]) - m2
        big = jnp.int32(2**30)
        bm8 = d[:, 0:128]
        for j in range(1, _KC // 128):
            bm8 = jnp.minimum(bm8, d[:, j * 128:(j + 1) * 128])
        ba8 = jnp.full((_T, 128), big, jnp.int32)
        for j in range(_KC // 128 - 1, -1, -1):
            eq = d[:, j * 128:(j + 1) * 128] == bm8
            ba8 = jnp.where(eq, cols0[:, j * 128:(j + 1) * 128], ba8)
        bm = jnp.min(bm8, axis=1)
        ba = jnp.min(jnp.where(bm8 == bm[:, None], ba8, big),
                     axis=1) + k * _KC
        upd = bm < mv
        return jnp.where(upd, bm, mv), jnp.where(upd, ba, mi)

    def half_min(k0):
        mv = jnp.full((_T,), jnp.inf, jnp.float32)
        mi = jnp.zeros((_T,), jnp.int32)
        for k in range(k0, k0 + _NKC // 2):
            mv, mi = chunk(k, (mv, mi))
        return mv, mi

    mv_a, mi_a = half_min(0)
    mv_b, mi_b = half_min(_NKC // 2)
    mva_q = mv_a.astype(jnp.bfloat16).astype(jnp.float32)
    upd = mv_b < mva_q
    mv = jnp.where(upd, mv_b, mv_a)
    mi = jnp.where(upd, mi_b, mi_a)
    idx_ref[0, 0, :] = mi
    cols = lax.broadcasted_iota(jnp.int32, (_T, _K), 1)
    enc_ref[...] = (cols == mi[:, None]).astype(jnp.float32)
    lp_ref[0, 0, 0] = jnp.sum(mv)


def _vq_argmin(flat, W, zsq3):
    return pl.pallas_call(
        _vq_argmin_body,
        grid=(_NT,),
        in_specs=[
            pl.BlockSpec((_T, _D), lambda t: (t, 0)),
            pl.BlockSpec((_K, _D), lambda t: (0, 0)),
            pl.BlockSpec((1, 1, _T), lambda t: (t, 0, 0)),
        ],
        out_specs=[
            pl.BlockSpec((1, 1, _T), lambda t: (t, 0, 0)),
            pl.BlockSpec((_T, _K), lambda t: (t, 0)),
            pl.BlockSpec((1, 1, 1), lambda t: (t, 0, 0),
                         memory_space=pltpu.SMEM),
        ],
        out_shape=[
            jax.ShapeDtypeStruct((_NT, 1, _T), jnp.int32),
            jax.ShapeDtypeStruct((_N, _K), jnp.float32),
            jax.ShapeDtypeStruct((_NT, 1, 1), jnp.float32),
        ],
    )(flat, W, zsq3)


def _gather_rows(W, idx3):
    info = plsc.get_sparse_core_info()
    nc, ns = info.num_cores, info.num_subcores
    nw = nc * ns
    bpw = _N // nw
    nch = bpw // 128
    mesh = plsc.VectorSubcoreMesh(core_axis_name="c", subcore_axis_name="s")

    @functools.partial(
        pl.kernel, mesh=mesh,
        out_type=jax.ShapeDtypeStruct((_N, _D), jnp.float32),
        scratch_types=[
            pltpu.VMEM((nch, 128), jnp.int32),
            pltpu.VMEM((bpw, _D), jnp.float32),
            pltpu.SemaphoreType.DMA,
        ],
    )
    def body(idx_hbm, w_hbm, out_hbm, idx_v, rows_v, sem):
        wid = lax.axis_index("s") * nc + lax.axis_index("c")
        base = wid * bpw
        pltpu.sync_copy(idx_hbm.at[wid], idx_v)
        copies = [
            pltpu.async_copy(w_hbm.at[idx_v.at[j]],
                             rows_v.at[pl.ds(j * 128, 128)], sem)
            for j in range(nch)
        ]
        for c in copies:
            c.wait()
        pltpu.sync_copy(rows_v, out_hbm.at[pl.ds(base, bpw)])

    return body(idx3, W)


def kernel(x, W):
    inputs = jnp.transpose(x, (0, 2, 3, 1))
    flat = inputs.reshape(-1, _D)
    zsq = jnp.sum(flat ** 2, axis=1)
    idx3, enc, lpart = _vq_argmin(flat, W * 2.0, zsq.reshape(_NT, 1, _T))
    nw = 32
    qflat = _gather_rows(W, idx3.reshape(nw, _N // nw // 128, 128))
    quantized = qflat.reshape(inputs.shape)
    l = jnp.sum(lpart) / (_N * _D)
    loss = l + _COMMIT * l
    return loss, jnp.transpose(quantized, (0, 3, 1, 2)), enc


# speedup vs baseline: 9.0449x; 1.3655x over previous
import functools

import jax
import jax.numpy as jnp
from jax import lax
from jax.experimental import pallas as pl
from jax.experimental.pallas import tpu as pltpu
from jax.experimental.pallas import tpu_sc as plsc

_K = 8192
_D = 256
_N = 8192
_T = 256
_KC = 1024
_NT = _N // _T
_NKC = _K // _KC
_COMMIT = 0.25


def _vq_argmin_body(z_ref, w2_ref, zsq_ref, wsq_ref, idx_ref, enc_ref, lp_ref):
    z = z_ref[...]
    zsq = zsq_ref[0, 0, :]
    cols0 = lax.broadcasted_iota(jnp.int32, (_T, _KC), 1)

    def chunk(k, carry):
        mv, mi = carry
        w2 = w2_ref[pl.ds(k * _KC, _KC), :]
        m2 = lax.dot_general(z, w2, (((1,), (1,)), ((), ())),
                             preferred_element_type=jnp.float32)
        wsq = wsq_ref[0, 0, pl.ds(k * _KC, _KC)]
        d = (zsq[:, None] + wsq[None, :]) - m2
        big = jnp.int32(2**30)
        bm8 = d[:, 0:128]
        for j in range(1, _KC // 128):
            bm8 = jnp.minimum(bm8, d[:, j * 128:(j + 1) * 128])
        ba8 = jnp.full((_T, 128), big, jnp.int32)
        for j in range(_KC // 128 - 1, -1, -1):
            eq = d[:, j * 128:(j + 1) * 128] == bm8
            ba8 = jnp.where(eq, cols0[:, j * 128:(j + 1) * 128], ba8)
        bm = jnp.min(bm8, axis=1)
        ba = jnp.min(jnp.where(bm8 == bm[:, None], ba8, big),
                     axis=1) + k * _KC
        upd = bm < mv
        return jnp.where(upd, bm, mv), jnp.where(upd, ba, mi)

    def half_min(k0):
        mv = jnp.full((_T,), jnp.inf, jnp.float32)
        mi = jnp.zeros((_T,), jnp.int32)
        for k in range(k0, k0 + _NKC // 2):
            mv, mi = chunk(k, (mv, mi))
        return mv, mi

    mv_a, mi_a = half_min(0)
    mv_b, mi_b = half_min(_NKC // 2)
    mva_q = mv_a.astype(jnp.bfloat16).astype(jnp.float32)
    upd = mv_b < mva_q
    mv = jnp.where(upd, mv_b, mv_a)
    mi = jnp.where(upd, mi_b, mi_a)
    idx_ref[0, 0, :] = mi
    cols = lax.broadcasted_iota(jnp.int32, (_T, _K), 1)
    enc_ref[...] = (cols == mi[:, None]).astype(jnp.float32)
    lp_ref[0, 0, 0] = jnp.sum(mv)


def _vq_argmin(flat, W, zsq3, wsq3):
    return pl.pallas_call(
        _vq_argmin_body,
        grid=(_NT,),
        in_specs=[
            pl.BlockSpec((_T, _D), lambda t: (t, 0)),
            pl.BlockSpec((_K, _D), lambda t: (0, 0)),
            pl.BlockSpec((1, 1, _T), lambda t: (t, 0, 0)),
            pl.BlockSpec((1, 1, _K), lambda t: (0, 0, 0)),
        ],
        out_specs=[
            pl.BlockSpec((1, 1, _T), lambda t: (t, 0, 0)),
            pl.BlockSpec((_T, _K), lambda t: (t, 0)),
            pl.BlockSpec((1, 1, 1), lambda t: (t, 0, 0),
                         memory_space=pltpu.SMEM),
        ],
        out_shape=[
            jax.ShapeDtypeStruct((_NT, 1, _T), jnp.int32),
            jax.ShapeDtypeStruct((_N, _K), jnp.float32),
            jax.ShapeDtypeStruct((_NT, 1, 1), jnp.float32),
        ],
    )(flat, W, zsq3, wsq3)


def _gather_rows(W, idx3):
    info = plsc.get_sparse_core_info()
    nc, ns = info.num_cores, info.num_subcores
    nw = nc * ns
    bpw = _N // nw
    nch = bpw // 128
    mesh = plsc.VectorSubcoreMesh(core_axis_name="c", subcore_axis_name="s")

    @functools.partial(
        pl.kernel, mesh=mesh,
        out_type=jax.ShapeDtypeStruct((_N, _D), jnp.float32),
        scratch_types=[
            pltpu.VMEM((nch, 128), jnp.int32),
            pltpu.VMEM((bpw, _D), jnp.float32),
            pltpu.SemaphoreType.DMA,
        ],
    )
    def body(idx_hbm, w_hbm, out_hbm, idx_v, rows_v, sem):
        wid = lax.axis_index("s") * nc + lax.axis_index("c")
        base = wid * bpw
        pltpu.sync_copy(idx_hbm.at[wid], idx_v)
        copies = [
            pltpu.async_copy(w_hbm.at[idx_v.at[j]],
                             rows_v.at[pl.ds(j * 128, 128)], sem)
            for j in range(nch)
        ]
        for c in copies:
            c.wait()
        pltpu.sync_copy(rows_v, out_hbm.at[pl.ds(base, bpw)])

    return body(idx3, W)


def kernel(x, W):
    inputs = jnp.transpose(x, (0, 2, 3, 1))
    flat = inputs.reshape(-1, _D)
    zsq = jnp.sum(flat ** 2, axis=1)
    wsq = jnp.sum(W ** 2, axis=1)
    idx3, enc, lpart = _vq_argmin(flat, W * 2.0, zsq.reshape(_NT, 1, _T),
                                  wsq.reshape(1, 1, _K))
    nw = 32
    qflat = _gather_rows(W, idx3.reshape(nw, _N // nw // 128, 128))
    quantized = qflat.reshape(inputs.shape)
    l = jnp.sum(lpart) / (_N * _D)
    loss = l + _COMMIT * l
    return loss, jnp.transpose(quantized, (0, 3, 1, 2)), enc
